# baseline (speedup 1.0000x reference)
import functools

import jax
import jax.numpy as jnp
from jax import lax
from jax.experimental import pallas as pl
from jax.experimental.pallas import tpu as pltpu
from jax.experimental.pallas import tpu_sc as plsc

_N = 10000
_E = 320000
_D = 128

_NW = 32
_C = 128
_KPW = 80
_EPAD = _NW * _KPW * _C
_RPW = _N // 16


def _aggregate_sc(feature, src_flat, dst3d, w_flat):
    mesh = plsc.VectorSubcoreMesh(core_axis_name="c", subcore_axis_name="s")

    @functools.partial(
        pl.kernel,
        mesh=mesh,
        out_type=jax.ShapeDtypeStruct((2, _N, _D), jnp.float32),
        scratch_types=[
            pltpu.VMEM((_C,), jnp.int32),
            pltpu.VMEM((_C,), jnp.int32),
            pltpu.VMEM((_KPW, _C), jnp.int32),
            pltpu.VMEM((_C,), jnp.float32),
            pltpu.VMEM((_C,), jnp.float32),
            pltpu.VMEM((_C, _D), jnp.float32),
            pltpu.VMEM((_C, _D), jnp.float32),
            pltpu.VMEM_SHARED((_N, _D), jnp.float32),
            pltpu.SemaphoreType.DMA,
            pltpu.SemaphoreType.DMA,
            pltpu.SemaphoreType.DMA,
            pltpu.SemaphoreType.DMA,
            pltpu.SemaphoreType.DMA,
            pltpu.SemaphoreType.DMA,
            pltpu.SemaphoreType.DMA,
            pltpu.SemaphoreType.DMA,
        ],
    )
    def body(feat_hbm, src_hbm, dst_hbm, w_hbm, out_hbm,
             src_a, src_b, dst_v, w_a, w_b, rows_a, rows_b, acc_sh,
             sem_ga, sem_gb, sem_sa, sem_sb, sem_wa, sem_wb,
             sem_ia, sem_ib):
        c = lax.axis_index("c")
        s = lax.axis_index("s")
        wkr = s * 2 + c
        ebase = wkr * _KPW * _C

        z16 = jnp.zeros((16,), jnp.float32)

        def _zrow(r, carry):
            for j in range(_D // 16):
                rows_a[r, pl.ds(j * 16, 16)] = z16
            return carry

        lax.fori_loop(0, _C, _zrow, 0)
        for j in range(_RPW // _C):
            pltpu.sync_copy(
                rows_a.at[pl.ds(0, _C)],
                acc_sh.at[pl.ds(s * _RPW + j * _C, _C)],
            )
        _zr = _RPW - (_RPW // _C) * _C
        pltpu.sync_copy(
            rows_a.at[pl.ds(0, _zr)],
            acc_sh.at[pl.ds(s * _RPW + (_RPW // _C) * _C, _zr)],
        )
        plsc.subcore_barrier()

        pltpu.sync_copy(dst_hbm.at[wkr], dst_v)

        def _w_refill(k, buf, sem):
            pltpu.async_copy(w_hbm.at[pl.ds(ebase + k * _C, _C)], buf, sem)

        def _w_wait(buf, sem):
            pltpu.make_async_copy(w_hbm.at[pl.ds(0, _C)], buf, sem).wait()

        def _src_refill(k, buf, sem):
            pltpu.async_copy(src_hbm.at[pl.ds(ebase + k * _C, _C)], buf, sem)

        def _src_wait(buf, sem):
            pltpu.make_async_copy(src_hbm.at[pl.ds(0, _C)], buf, sem).wait()

        def _scale(buf, w_buf):
            def _grp(g, carry2):
                w16 = w_buf[pl.ds(g * 16, 16)]
                for i in range(16):
                    wspl = lax.gather(
                        w16, jnp.full((16, 1), i, jnp.int32),
                        lax.GatherDimensionNumbers(
                            offset_dims=(), collapsed_slice_dims=(0,),
                            start_index_map=(0,)),
                        (1,), mode=lax.GatherScatterMode.PROMISE_IN_BOUNDS)
                    e = g * 16 + i
                    for j in range(_D // 16):
                        sl = pl.ds(j * 16, 16)
                        buf[e, sl] = buf[e, sl] * wspl
                return carry2

            lax.fori_loop(0, _C // 16, _grp, 0)

        def _gather_start(idx, buf, sem):
            pltpu.async_copy(feat_hbm.at[idx.at[pl.ds(0, _C)]], buf, sem)

        def _gather_wait(buf, sem):
            pltpu.make_async_copy(
                feat_hbm.at[src_a.at[pl.ds(0, _C)]], buf, sem).wait()

        def _scatter_start(k, buf, sem):
            pltpu.async_copy(buf, acc_sh.at[dst_v.at[k]], sem, add=True)

        def _scatter_wait(buf, sem):
            pltpu.make_async_copy(buf, acc_sh.at[dst_v.at[0]], sem).wait()

        pltpu.sync_copy(src_hbm.at[pl.ds(ebase, _C)], src_a)
        _src_refill(1, src_b, sem_ib)
        _w_refill(0, w_a, sem_wa)
        _w_refill(1, w_b, sem_wb)
        _gather_start(src_a, rows_a, sem_ga)

        def _pair(p, carry):
            k0 = 2 * p
            k2 = lax.rem(k0 + 2, _KPW)
            k3 = lax.rem(k0 + 3, _KPW)
            _gather_wait(rows_a, sem_ga)

            @pl.when(p > 0)
            def _():
                _scatter_wait(rows_b, sem_sb)

            _src_refill(k2, src_a, sem_ia)
            _src_wait(src_b, sem_ib)
            _gather_start(src_b, rows_b, sem_gb)
            _w_wait(w_a, sem_wa)
            _scale(rows_a, w_a)
            _scatter_start(k0, rows_a, sem_sa)
            _w_refill(k2, w_a, sem_wa)
            _gather_wait(rows_b, sem_gb)
            _src_refill(k3, src_b, sem_ib)
            _scatter_wait(rows_a, sem_sa)
            _src_wait(src_a, sem_ia)
            _gather_start(src_a, rows_a, sem_ga)
            _w_wait(w_b, sem_wb)
            _scale(rows_b, w_b)
            _scatter_start(k0 + 1, rows_b, sem_sb)
            _w_refill(k3, w_b, sem_wb)
            return carry

        lax.fori_loop(0, _KPW // 2, _pair, 0)
        _gather_wait(rows_a, sem_ga)
        _scatter_wait(rows_b, sem_sb)
        _w_wait(w_a, sem_wa)
        _w_wait(w_b, sem_wb)
        _src_wait(src_b, sem_ib)
        plsc.subcore_barrier()

        nchunks = _N // 80
        for j in range(8):
            k = s + 16 * j

            @pl.when(k < nchunks)
            def _():
                r = k * 80
                pltpu.sync_copy(acc_sh.at[pl.ds(r, 80)],
                                rows_a.at[pl.ds(0, 80)])
                pltpu.sync_copy(rows_a.at[pl.ds(0, 80)],
                                out_hbm.at[c, pl.ds(r, 80)])

    return body(feature, src_flat, dst3d, w_flat)


def _dense_body(p0_ref, p1_ref, w_ref, b_ref, g_ref, bt_ref, o_ref):
    h = p0_ref[...] + p1_ref[...]
    y = lax.dot_general(h, w_ref[...], (((1,), (1,)), ((), ())),
                        preferred_element_type=jnp.float32)
    y = jnp.maximum(y + b_ref[...], 0.0)
    mean = jnp.mean(y, axis=0, keepdims=True)
    var = jnp.mean(jnp.square(y - mean), axis=0, keepdims=True)
    o_ref[...] = (y - mean) / jnp.sqrt(var + 1e-5) * g_ref[...] + bt_ref[...]


def kernel(feature, edge_index, edge_weight, W, b, gamma, beta):
    src = edge_index[0].astype(jnp.int32)
    dst = edge_index[1].astype(jnp.int32)
    w = edge_weight.reshape(_E).astype(jnp.float32)
    pad = _EPAD - _E
    src_flat = jnp.concatenate([src, jnp.zeros((pad,), jnp.int32)])
    dst3d = jnp.concatenate([dst, jnp.zeros((pad,), jnp.int32)]).reshape(
        _NW, _KPW, _C)
    w_flat = jnp.concatenate([w, jnp.zeros((pad,), jnp.float32)])

    partials = _aggregate_sc(feature, src_flat, dst3d, w_flat)

    out = pl.pallas_call(
        _dense_body,
        out_shape=jax.ShapeDtypeStruct((_N, _D), jnp.float32),
    )(partials[0], partials[1], W,
      b.reshape(1, _D), gamma.reshape(1, _D), beta.reshape(1, _D))
    return out


# speedup vs baseline: 1.0226x; 1.0226x over previous
import functools

import jax
import jax.numpy as jnp
from jax import lax
from jax.experimental import pallas as pl
from jax.experimental.pallas import tpu as pltpu
from jax.experimental.pallas import tpu_sc as plsc

_N = 10000
_E = 320000
_D = 128

_NW = 32
_NB = 4
_C = 64
_KPW = 160
_EPAD = _NW * _KPW * _C
_RPW = _N // 16
_WCH = 80 if _C >= 80 else 40


def _aggregate_sc(feature, src_flat, dst_flat, w_flat):
    mesh = plsc.VectorSubcoreMesh(core_axis_name="c", subcore_axis_name="s")

    scratch = (
        [pltpu.VMEM((_C,), jnp.int32) for _ in range(_NB)]
        + [pltpu.VMEM((_C,), jnp.float32) for _ in range(_NB)]
        + [pltpu.VMEM((_C, _D), jnp.float32) for _ in range(_NB)]
        + [pltpu.VMEM((_KPW * _C,), jnp.int32)]
        + [pltpu.VMEM_SHARED((_N, _D), jnp.float32)]
        + [pltpu.SemaphoreType.DMA] * (4 * _NB)
    )

    @functools.partial(
        pl.kernel,
        mesh=mesh,
        out_type=jax.ShapeDtypeStruct((2, _N, _D), jnp.float32),
        scratch_types=scratch,
    )
    def body(feat_hbm, src_hbm, dst_hbm, w_hbm, out_hbm, *sc):
        src_v = sc[0:_NB]
        w_v = sc[_NB:2 * _NB]
        rows = sc[2 * _NB:3 * _NB]
        dst_v = sc[3 * _NB]
        acc_sh = sc[3 * _NB + 1]
        sem_g = sc[3 * _NB + 2:3 * _NB + 2 + _NB]
        sem_s = sc[3 * _NB + 2 + _NB:3 * _NB + 2 + 2 * _NB]
        sem_w = sc[3 * _NB + 2 + 2 * _NB:3 * _NB + 2 + 3 * _NB]
        sem_i = sc[3 * _NB + 2 + 3 * _NB:3 * _NB + 2 + 4 * _NB]

        c = lax.axis_index("c")
        s = lax.axis_index("s")
        wkr = s * 2 + c
        ebase = wkr * _KPW * _C

        z16 = jnp.zeros((16,), jnp.float32)

        def _zrow(r, carry):
            for j in range(_D // 16):
                rows[0][r, pl.ds(j * 16, 16)] = z16
            return carry

        lax.fori_loop(0, _C, _zrow, 0)
        for j in range(_RPW // _C):
            pltpu.sync_copy(
                rows[0].at[pl.ds(0, _C)],
                acc_sh.at[pl.ds(s * _RPW + j * _C, _C)],
            )
        _zr = _RPW - (_RPW // _C) * _C
        pltpu.sync_copy(
            rows[0].at[pl.ds(0, _zr)],
            acc_sh.at[pl.ds(s * _RPW + (_RPW // _C) * _C, _zr)],
        )
        plsc.subcore_barrier()

        pltpu.sync_copy(dst_hbm.at[pl.ds(ebase, _KPW * _C)], dst_v)

        def _w_refill(k, b):
            pltpu.async_copy(w_hbm.at[pl.ds(ebase + k * _C, _C)],
                             w_v[b], sem_w[b])

        def _w_wait(b):
            pltpu.make_async_copy(w_hbm.at[pl.ds(0, _C)],
                                  w_v[b], sem_w[b]).wait()

        def _src_refill(k, b):
            pltpu.async_copy(src_hbm.at[pl.ds(ebase + k * _C, _C)],
                             src_v[b], sem_i[b])

        def _src_wait(b):
            pltpu.make_async_copy(src_hbm.at[pl.ds(0, _C)],
                                  src_v[b], sem_i[b]).wait()

        def _scale(b):
            buf = rows[b]
            w_buf = w_v[b]

            def _grp(g, carry2):
                w16 = w_buf[pl.ds(g * 16, 16)]
                for i in range(16):
                    wspl = lax.gather(
                        w16, jnp.full((16, 1), i, jnp.int32),
                        lax.GatherDimensionNumbers(
                            offset_dims=(), collapsed_slice_dims=(0,),
                            start_index_map=(0,)),
                        (1,), mode=lax.GatherScatterMode.PROMISE_IN_BOUNDS)
                    e = g * 16 + i
                    for j in range(_D // 16):
                        sl = pl.ds(j * 16, 16)
                        buf[e, sl] = buf[e, sl] * wspl
                return carry2

            lax.fori_loop(0, _C // 16, _grp, 0)

        def _gather_start(b):
            pltpu.async_copy(feat_hbm.at[src_v[b].at[pl.ds(0, _C)]],
                             rows[b], sem_g[b])

        def _gather_wait(b):
            pltpu.make_async_copy(
                feat_hbm.at[src_v[0].at[pl.ds(0, _C)]],
                rows[b], sem_g[b]).wait()

        def _scatter_start(k, b):
            pltpu.async_copy(rows[b], acc_sh.at[dst_v.at[pl.ds(k * _C, _C)]],
                             sem_s[b], add=True)

        def _scatter_wait(b):
            pltpu.make_async_copy(rows[b],
                                  acc_sh.at[dst_v.at[pl.ds(0, _C)]],
                                  sem_s[b]).wait()

        for b in range(_NB):
            pltpu.sync_copy(src_hbm.at[pl.ds(ebase + b * _C, _C)], src_v[b])
            _w_refill(b, b)
            _gather_start(b)

        def _round(r, carry):
            for b in range(_NB):
                k = r * _NB + b
                kn = lax.rem(k + _NB, _KPW)
                _gather_wait(b)
                _src_refill(kn, b)
                _w_wait(b)
                _scale(b)
                _w_refill(kn, b)
                _scatter_start(k, b)
                _scatter_wait(b)
                _src_wait(b)
                _gather_start(b)
            return carry

        lax.fori_loop(0, _KPW // _NB, _round, 0)
        for b in range(_NB):
            _gather_wait(b)
            _w_wait(b)
        plsc.subcore_barrier()

        nchunks = _N // _WCH
        for j in range((nchunks + 15) // 16):
            k = s + 16 * j

            @pl.when(k < nchunks)
            def _():
                r = k * _WCH
                pltpu.sync_copy(acc_sh.at[pl.ds(r, _WCH)],
                                rows[0].at[pl.ds(0, _WCH)])
                pltpu.sync_copy(rows[0].at[pl.ds(0, _WCH)],
                                out_hbm.at[c, pl.ds(r, _WCH)])

    return body(feature, src_flat, dst_flat, w_flat)


def _dense_body(p0_ref, p1_ref, w_ref, b_ref, g_ref, bt_ref, o_ref):
    h = p0_ref[...] + p1_ref[...]
    y = lax.dot_general(h, w_ref[...], (((1,), (1,)), ((), ())),
                        preferred_element_type=jnp.float32)
    y = jnp.maximum(y + b_ref[...], 0.0)
    mean = jnp.mean(y, axis=0, keepdims=True)
    var = jnp.mean(jnp.square(y - mean), axis=0, keepdims=True)
    o_ref[...] = (y - mean) / jnp.sqrt(var + 1e-5) * g_ref[...] + bt_ref[...]


def kernel(feature, edge_index, edge_weight, W, b, gamma, beta):
    src = edge_index[0].astype(jnp.int32)
    dst = edge_index[1].astype(jnp.int32)
    w = edge_weight.reshape(_E).astype(jnp.float32)
    pad = _EPAD - _E
    src_flat = jnp.concatenate([src, jnp.zeros((pad,), jnp.int32)])
    dst_flat = jnp.concatenate([dst, jnp.zeros((pad,), jnp.int32)])
    w_flat = jnp.concatenate([w, jnp.zeros((pad,), jnp.float32)])

    partials = _aggregate_sc(feature, src_flat, dst_flat, w_flat)

    out = pl.pallas_call(
        _dense_body,
        out_shape=jax.ShapeDtypeStruct((_N, _D), jnp.float32),
    )(partials[0], partials[1], W,
      b.reshape(1, _D), gamma.reshape(1, _D), beta.reshape(1, _D))
    return out
